# Initial kernel scaffold; baseline (speedup 1.0000x reference)
import jax
import jax.numpy as jnp
from jax.experimental import pallas as pl


def kernel(edges, poses, nodes):
    raise NotImplementedError("write your pallas kernel here")


# speedup vs baseline: 5.7244x; 5.7244x over previous
import functools

import jax
import jax.numpy as jnp
from jax import lax
from jax.experimental import pallas as pl
from jax.experimental.pallas import tpu as pltpu
from jax.experimental.pallas import tpu_sc as plsc

NC = 2
NS = 16
NW = NC * NS
L = 16

_ATAN_C = (
    0.9999993278352406,
    -0.33326374521881463,
    0.1987987215570719,
    -0.13480405607533819,
    0.08374155654488141,
    -0.036898629246094994,
    0.007825482945455486,
)


def _f32(v):
    return jnp.full((L,), v, dtype=jnp.float32)


def _rsqrt(x):
    y = jnp.int32(0x5F3759DF) - (plsc.bitcast(x, jnp.int32) >> 1)
    r = plsc.bitcast(y, jnp.float32)
    half_x = 0.5 * x
    for _ in range(3):
        r = r * (1.5 - half_x * r * r)
    return r


def _atan2_pos(n, w):
    a = jnp.minimum(n, w)
    b = jnp.maximum(n, w)
    t = a / jnp.maximum(b, 1e-30)
    z = t * t
    p = _f32(_ATAN_C[-1])
    for c in reversed(_ATAN_C[:-1]):
        p = p * z + c
    p = p * t
    return jnp.where(n > w, 1.5707963267948966 - p, p)


def _cross(a, b):
    return (
        a[1] * b[2] - a[2] * b[1],
        a[2] * b[0] - a[0] * b[2],
        a[0] * b[1] - a[1] * b[0],
    )


def _qrot(q, v):
    qv = q[:3]
    tx, ty, tz = _cross(qv, v)
    tx, ty, tz = 2.0 * tx, 2.0 * ty, 2.0 * tz
    cx, cy, cz = _cross(qv, (tx, ty, tz))
    return (v[0] + q[3] * tx + cx, v[1] + q[3] * ty + cy, v[2] + q[3] * tz + cz)


def _qmul(q, r):
    qx, qy, qz, qw = q
    rx, ry, rz, rw = r
    return (
        qw * rx + qx * rw + qy * rz - qz * ry,
        qw * ry - qx * rz + qy * rw + qz * rx,
        qw * rz + qx * ry - qy * rx + qz * rw,
        qw * rw - qx * rx - qy * ry - qz * rz,
    )


def _edge_math(t1, q1, t2, q2, tp, qp):
    q1i = (-q1[0], -q1[1], -q1[2], q1[3])
    t1r = _qrot(q1i, t1)
    t1i = (-t1r[0], -t1r[1], -t1r[2])
    qa = _qmul(q1i, q2)
    t2r = _qrot(q1i, t2)
    ta = (t1i[0] + t2r[0], t1i[1] + t2r[1], t1i[2] + t2r[2])
    qpi = (-qp[0], -qp[1], -qp[2], qp[3])
    tpr = _qrot(qpi, tp)
    tpi = (-tpr[0], -tpr[1], -tpr[2])
    qe = _qmul(qa, qpi)
    tbr = _qrot(qa, tpi)
    t = (ta[0] + tbr[0], ta[1] + tbr[1], ta[2] + tbr[2])
    sgn = jnp.where(qe[3] < 0.0, _f32(-1.0), _f32(1.0))
    qx, qy, qz, qw = sgn * qe[0], sgn * qe[1], sgn * qe[2], sgn * qe[3]
    n2 = qx * qx + qy * qy + qz * qz
    xx = n2 + 1e-24
    r = _rsqrt(xx)
    n = xx * r
    theta = 2.0 * _atan2_pos(n, qw)
    small = n < 1e-6
    f = jnp.where(small, 2.0 / jnp.maximum(qw, 1e-6), theta * r)
    phi = (f * qx, f * qy, f * qz)
    nn = n * n
    inv_h2 = 1.0 / (nn + qw * qw)
    omc = 2.0 * nn * inv_h2
    st = 2.0 * n * qw * inv_h2
    denom = 2.0 * (theta * theta) * omc
    safe_denom = jnp.where(small, _f32(1.0), denom)
    coef = jnp.where(small, _f32(1.0 / 12.0), (2.0 * omc - theta * st) / safe_denom)
    px, py, pz = _cross(phi, t)
    cx, cy, cz = _cross(phi, (px, py, pz))
    tau = (
        t[0] - 0.5 * px + coef * cx,
        t[1] - 0.5 * py + coef * cy,
        t[2] - 0.5 * pz + coef * cz,
    )
    return tau + phi


def _make_kernel(E, N, C):
    per_w = E // NW
    nchunks = per_w // C
    mesh = plsc.VectorSubcoreMesh(core_axis_name="c", subcore_axis_name="s")

    @functools.partial(
        pl.kernel,
        out_type=jax.ShapeDtypeStruct((E, 6), jnp.float32),
        mesh=mesh,
        compiler_params=pltpu.CompilerParams(
            needs_layout_passes=False, use_tc_tiling_on_sc=False
        ),
        scratch_types=[
            pltpu.VMEM((C,), jnp.int32),
            pltpu.VMEM((C,), jnp.int32),
            pltpu.VMEM((C, 8), jnp.float32),
            pltpu.VMEM((C, 8), jnp.float32),
            pltpu.VMEM((C, 7), jnp.float32),
            pltpu.VMEM((C, 6), jnp.float32),
            pltpu.SemaphoreType.DMA,
        ],
    )
    def k(idx1_hbm, idx2_hbm, poses_hbm, nodes_hbm, out_hbm,
          idx1_v, idx2_v, n1_v, n2_v, p_v, o_v, sem):
        wid = lax.axis_index("s") * NC + lax.axis_index("c")
        wbase = wid * per_w
        iota = lax.iota(jnp.int32, L)
        cols = [jnp.full((L,), f, dtype=jnp.int32) for f in range(7)]

        def chunk(g, carry):
            base = wbase + g * C
            pltpu.sync_copy(idx1_hbm.at[pl.ds(base, C)], idx1_v)
            pltpu.sync_copy(idx2_hbm.at[pl.ds(base, C)], idx2_v)
            cp1 = pltpu.async_copy(nodes_hbm.at[idx1_v], n1_v, sem)
            cp2 = pltpu.async_copy(nodes_hbm.at[idx2_v], n2_v, sem)
            pltpu.sync_copy(poses_hbm.at[pl.ds(base, C)], p_v)
            cp1.wait()
            cp2.wait()

            def group(i, c2):
                rows = i * L + iota
                g1 = [plsc.load_gather(n1_v, [rows, cols[f]]) for f in range(7)]
                g2 = [plsc.load_gather(n2_v, [rows, cols[f]]) for f in range(7)]
                gp = [plsc.load_gather(p_v, [rows, cols[f]]) for f in range(7)]
                res = _edge_math(
                    tuple(g1[:3]), tuple(g1[3:]),
                    tuple(g2[:3]), tuple(g2[3:]),
                    tuple(gp[:3]), tuple(gp[3:]),
                )
                for f in range(6):
                    plsc.store_scatter(o_v, [rows, cols[f]], res[f])
                return c2

            lax.fori_loop(0, C // L, group, 0)
            pltpu.sync_copy(o_v, out_hbm.at[pl.ds(base, C)])
            return carry

        lax.fori_loop(0, nchunks, chunk, 0)

    return k


@jax.jit
def kernel(edges, poses, nodes):
    E = edges.shape[0]
    N = nodes.shape[0]
    idx = edges.astype(jnp.int32)
    idx1 = idx[:, 0]
    idx2 = idx[:, 1]
    nodes_p = jnp.pad(nodes.astype(jnp.float32), ((0, 0), (0, 1)))
    k = _make_kernel(E, N, C=800)
    return k(idx1, idx2, poses.astype(jnp.float32), nodes_p)


# speedup vs baseline: 6.4255x; 1.1225x over previous
import functools

import jax
import jax.numpy as jnp
from jax import lax
from jax.experimental import pallas as pl
from jax.experimental.pallas import tpu as pltpu
from jax.experimental.pallas import tpu_sc as plsc

NC = 2
NS = 16
NW = NC * NS
L = 16

_ATAN_C = (
    0.9999993278352406,
    -0.33326374521881463,
    0.1987987215570719,
    -0.13480405607533819,
    0.08374155654488141,
    -0.036898629246094994,
    0.007825482945455486,
)


def _f32(v):
    return jnp.full((L,), v, dtype=jnp.float32)


def _rsqrt(x):
    y = jnp.int32(0x5F3759DF) - (plsc.bitcast(x, jnp.int32) >> 1)
    r = plsc.bitcast(y, jnp.float32)
    half_x = 0.5 * x
    for _ in range(3):
        r = r * (1.5 - half_x * r * r)
    return r


def _atan2_pos(n, w):
    a = jnp.minimum(n, w)
    b = jnp.maximum(n, w)
    t = a / jnp.maximum(b, 1e-30)
    z = t * t
    p = _f32(_ATAN_C[-1])
    for c in reversed(_ATAN_C[:-1]):
        p = p * z + c
    p = p * t
    return jnp.where(n > w, 1.5707963267948966 - p, p)


def _cross(a, b):
    return (
        a[1] * b[2] - a[2] * b[1],
        a[2] * b[0] - a[0] * b[2],
        a[0] * b[1] - a[1] * b[0],
    )


def _qrot(q, v):
    qv = q[:3]
    tx, ty, tz = _cross(qv, v)
    tx, ty, tz = 2.0 * tx, 2.0 * ty, 2.0 * tz
    cx, cy, cz = _cross(qv, (tx, ty, tz))
    return (v[0] + q[3] * tx + cx, v[1] + q[3] * ty + cy, v[2] + q[3] * tz + cz)


def _qmul(q, r):
    qx, qy, qz, qw = q
    rx, ry, rz, rw = r
    return (
        qw * rx + qx * rw + qy * rz - qz * ry,
        qw * ry - qx * rz + qy * rw + qz * rx,
        qw * rz + qx * ry - qy * rx + qz * rw,
        qw * rw - qx * rx - qy * ry - qz * rz,
    )


def _edge_math(t1, q1, t2, q2, tp, qp):
    q1i = (-q1[0], -q1[1], -q1[2], q1[3])
    d = (t2[0] - t1[0], t2[1] - t1[1], t2[2] - t1[2])
    ta = _qrot(q1i, d)
    qa = _qmul(q1i, q2)
    qpi = (-qp[0], -qp[1], -qp[2], qp[3])
    qe = _qmul(qa, qpi)
    tbr = _qrot(qe, tp)
    t = (ta[0] - tbr[0], ta[1] - tbr[1], ta[2] - tbr[2])
    sgn = jnp.where(qe[3] < 0.0, _f32(-1.0), _f32(1.0))
    qx, qy, qz, qw = sgn * qe[0], sgn * qe[1], sgn * qe[2], sgn * qe[3]
    n2 = qx * qx + qy * qy + qz * qz
    xx = n2 + 1e-24
    r = _rsqrt(xx)
    n = xx * r
    theta = 2.0 * _atan2_pos(n, qw)
    small = n < 1e-6
    f = jnp.where(small, 2.0 / jnp.maximum(qw, 1e-6), theta * r)
    phi = (f * qx, f * qy, f * qz)
    nn = n * n
    inv_h2 = 1.0 / (nn + qw * qw)
    omc = 2.0 * nn * inv_h2
    st = 2.0 * n * qw * inv_h2
    denom = 2.0 * (theta * theta) * omc
    safe_denom = jnp.where(small, _f32(1.0), denom)
    coef = jnp.where(small, _f32(1.0 / 12.0), (2.0 * omc - theta * st) / safe_denom)
    px, py, pz = _cross(phi, t)
    cx, cy, cz = _cross(phi, (px, py, pz))
    tau = (
        t[0] - 0.5 * px + coef * cx,
        t[1] - 0.5 * py + coef * cy,
        t[2] - 0.5 * pz + coef * cz,
    )
    return tau + phi


def _make_kernel(E, N, C):
    per_w = E // NW
    nchunks = per_w // C
    assert nchunks >= 3
    mesh = plsc.VectorSubcoreMesh(core_axis_name="c", subcore_axis_name="s")

    @functools.partial(
        pl.kernel,
        out_type=jax.ShapeDtypeStruct((E, 6), jnp.float32),
        mesh=mesh,
        compiler_params=pltpu.CompilerParams(
            needs_layout_passes=False, use_tc_tiling_on_sc=False
        ),
        scratch_types=[
            [pltpu.VMEM((C,), jnp.int32)] * 2,
            [pltpu.VMEM((C,), jnp.int32)] * 2,
            [pltpu.VMEM((C, 8), jnp.float32)] * 2,
            [pltpu.VMEM((C, 8), jnp.float32)] * 2,
            [pltpu.VMEM((C, 7), jnp.float32)] * 2,
            [pltpu.VMEM((C, 6), jnp.float32)] * 2,
            [pltpu.SemaphoreType.DMA] * 2,
            [pltpu.SemaphoreType.DMA] * 2,
            [pltpu.SemaphoreType.DMA] * 2,
        ],
    )
    def k(idx1_hbm, idx2_hbm, poses_hbm, nodes_hbm, out_hbm,
          i1, i2, n1, n2, p, o, sidx, sgat, sout):
        wid = lax.axis_index("s") * NC + lax.axis_index("c")
        wbase = wid * per_w
        iota = lax.iota(jnp.int32, L)
        cols = [jnp.full((L,), f, dtype=jnp.int32) for f in range(7)]

        def issue_idx(g, b):
            base = wbase + g * C
            pltpu.async_copy(idx1_hbm.at[pl.ds(base, C)], i1[b], sidx[b])
            pltpu.async_copy(idx2_hbm.at[pl.ds(base, C)], i2[b], sidx[b])

        def wait_idx(b):
            pltpu.make_async_copy(idx1_hbm.at[pl.ds(0, C)], i1[b], sidx[b]).wait()
            pltpu.make_async_copy(idx2_hbm.at[pl.ds(0, C)], i2[b], sidx[b]).wait()

        def issue_gat(g, b):
            base = wbase + g * C
            pltpu.async_copy(nodes_hbm.at[i1[b]], n1[b], sgat[b])
            pltpu.async_copy(nodes_hbm.at[i2[b]], n2[b], sgat[b])
            pltpu.async_copy(poses_hbm.at[pl.ds(base, C)], p[b], sgat[b])

        def wait_gat(b):
            pltpu.make_async_copy(nodes_hbm.at[pl.ds(0, C)], n1[b], sgat[b]).wait()
            pltpu.make_async_copy(nodes_hbm.at[pl.ds(0, C)], n2[b], sgat[b]).wait()
            pltpu.make_async_copy(poses_hbm.at[pl.ds(0, C)], p[b], sgat[b]).wait()

        def issue_out(g, b):
            base = wbase + g * C
            pltpu.async_copy(o[b], out_hbm.at[pl.ds(base, C)], sout[b])

        def wait_out(b):
            pltpu.make_async_copy(o[b], out_hbm.at[pl.ds(0, C)], sout[b]).wait()

        def compute(b):
            def group(i, c2):
                rows = i * L + iota
                g1 = [plsc.load_gather(n1[b], [rows, cols[f]]) for f in range(7)]
                g2 = [plsc.load_gather(n2[b], [rows, cols[f]]) for f in range(7)]
                gp = [plsc.load_gather(p[b], [rows, cols[f]]) for f in range(7)]
                res = _edge_math(
                    tuple(g1[:3]), tuple(g1[3:]),
                    tuple(g2[:3]), tuple(g2[3:]),
                    tuple(gp[:3]), tuple(gp[3:]),
                )
                for f in range(6):
                    plsc.store_scatter(o[b], [rows, cols[f]], res[f])
                return c2

            lax.fori_loop(0, C // L, group, 0)

        def step(g, b, b1):
            wait_gat(b)

            @pl.when(g + 1 < nchunks)
            def _():
                wait_idx(b1)
                issue_gat(g + 1, b1)

            @pl.when(g + 2 < nchunks)
            def _():
                issue_idx(g + 2, b)

            @pl.when(g >= 2)
            def _():
                wait_out(b)

            compute(b)
            issue_out(g, b)

        issue_idx(0, 0)
        issue_idx(1, 1)
        wait_idx(0)
        issue_gat(0, 0)

        def pair(h, carry):
            g = h * 2
            step(g, 0, 1)
            step(g + 1, 1, 0)
            return carry

        lax.fori_loop(0, nchunks // 2, pair, 0)
        if nchunks % 2:
            step(nchunks - 1, 0, 1)
        wait_out((nchunks - 2) % 2)
        wait_out((nchunks - 1) % 2)

    return k


@jax.jit
def kernel(edges, poses, nodes):
    E = edges.shape[0]
    N = nodes.shape[0]
    idx = edges.astype(jnp.int32)
    idx1 = idx[:, 0]
    idx2 = idx[:, 1]
    nodes_p = jnp.pad(nodes.astype(jnp.float32), ((0, 0), (0, 1)))
    k = _make_kernel(E, N, C=800)
    return k(idx1, idx2, poses.astype(jnp.float32), nodes_p)
